# baseline (speedup 1.0000x reference)
import functools

import jax
import jax.numpy as jnp
from jax import lax
from jax.experimental import pallas as pl
from jax.experimental.pallas import tpu as pltpu
from jax.experimental.pallas import tpu_sc as plsc

_B, _S, _D = 4, 4096, 2048
_SURV = _S - (_S + 2) // 3
_NC, _NS = 2, 16
_CH = 24
_PARTS = (768, 768, 768, 426)


def _make_part(j0, nrows):
    nblocks, rem = divmod(nrows, 8)
    q, r2 = divmod(nblocks, 8)
    rows_q = 8 * q
    full = rows_q // 24
    assert rows_q % 24 == 0 and full >= 2 and r2 <= 7
    idxcap = -(-(rows_q + 8 + rem) // 16) * 16

    scratch = [
        pltpu.VMEM((idxcap,), jnp.int32),
        pltpu.VMEM((_CH, _D), jnp.float32),
        pltpu.VMEM((_CH, _D), jnp.float32),
        pltpu.VMEM((8, _D), jnp.float32),
        pltpu.VMEM((max(rem, 1), _D), jnp.float32),
        pltpu.SemaphoreType.DMA,
        pltpu.SemaphoreType.DMA,
        pltpu.SemaphoreType.DMA,
    ]

    @functools.partial(
        pl.kernel,
        mesh=plsc.VectorSubcoreMesh(core_axis_name="c", subcore_axis_name="s"),
        out_type=jax.ShapeDtypeStruct((_B, nrows, _D), jnp.float32),
        scratch_types=scratch,
    )
    def _part(table, out, idx_v, buf0, buf1, tb8, tbr, g0, g1, tsem):
        wid = lax.axis_index("s") * _NC + lax.axis_index("c")
        b = wid // 8
        o = wid % 8
        base = o * rows_q + 8 * jnp.minimum(o, r2)
        lanes = lax.iota(jnp.int32, 16)
        for i in range(idxcap // 16):
            j = j0 + base + i * 16 + lanes
            src = j + (j >> 1) + 1
            idx_v[pl.ds(i * 16, 16)] = jnp.minimum(src, _S - 1)

        plane = table.at[b]
        bufs = (buf0, buf1)
        gsems = (g0, g1)
        copies = [
            pltpu.async_copy(plane.at[idx_v.at[pl.ds(0, _CH)]], buf0, g0),
            pltpu.async_copy(plane.at[idx_v.at[pl.ds(_CH, _CH)]], buf1, g1),
        ]
        for t in range(full):
            s = t % 2
            copies[s].wait()
            pltpu.sync_copy(bufs[s], out.at[b, pl.ds(base + t * _CH, _CH)])
            nxt = t + 2
            if nxt < full:
                copies[s] = pltpu.async_copy(
                    plane.at[idx_v.at[pl.ds(nxt * _CH, _CH)]], bufs[s], gsems[s])

        if r2 > 0:
            @pl.when(o < r2)
            def _extra8():
                pltpu.async_copy(
                    plane.at[idx_v.at[pl.ds(rows_q, 8)]], tb8, tsem).wait()
                pltpu.sync_copy(tb8, out.at[b, pl.ds(base + rows_q, 8)])

        if rem > 0:
            @pl.when(o == 7)
            def _tail():
                pltpu.async_copy(
                    plane.at[idx_v.at[pl.ds(rows_q, rem)]], tbr, tsem).wait()
                pltpu.sync_copy(tbr, out.at[b, pl.ds(nrows - rem, rem)])

    return _part


_PART_FNS = []
_j0 = 0
for _R in _PARTS:
    _PART_FNS.append(_make_part(_j0, _R))
    _j0 += _R
assert _j0 == _SURV


def kernel(hidden_states):
    parts = [fn(hidden_states) for fn in _PART_FNS]
    return jnp.concatenate(parts, axis=1)
